# Initial kernel scaffold; baseline (speedup 1.0000x reference)
import jax
import jax.numpy as jnp
from jax.experimental import pallas as pl


def kernel(coords, params):
    raise NotImplementedError("write your pallas kernel here")


# speedup vs baseline: 7.5275x; 7.5275x over previous
import functools

import jax
import jax.numpy as jnp
from jax import lax
from jax.experimental import pallas as pl
from jax.experimental.pallas import tpu as pltpu
from jax.experimental.pallas import tpu_sc as plsc

N_POINTS = 131072
NLVL = 16
F = 4
NW = 32
PW = N_POINTS // NW
C = 32
D = 16
R_USED = 1 << 19
G = C // 16
NCHUNK = PW // C
B = NLVL * 8 * C

_P1 = -1640531535
_P2 = 805459861
_XMASK = (1 << 19) - 1

_mesh = plsc.VectorSubcoreMesh(core_axis_name="c", subcore_axis_name="s")


@functools.partial(
    pl.kernel,
    mesh=_mesh,
    out_type=jax.ShapeDtypeStruct((N_POINTS * NLVL * F,), jnp.float32),
    scratch_types=[
        pltpu.VMEM((C,), jnp.float32),
        pltpu.VMEM((C,), jnp.float32),
        pltpu.VMEM((C,), jnp.float32),
        pltpu.VMEM((B,), jnp.int32),
        pltpu.VMEM((B,), jnp.float32),
        pltpu.VMEM((B, D), jnp.float32),
        pltpu.VMEM((C * NLVL * F,), jnp.float32),
        pltpu.SemaphoreType.DMA,
    ],
    compiler_params=pltpu.CompilerParams(
        needs_layout_passes=False, use_tc_tiling_on_sc=False),
)
def _hashgrid_sc(cx_h, cy_h, cz_h, table_h, out_h,
                 cx, cy, cz, idx_b, wgt_b, gath_b, out_b, sem):
    wid = lax.axis_index("s") * 2 + lax.axis_index("c")
    base = wid * PW

    iota = lax.iota(jnp.int32, 16)
    iota_out = iota * (NLVL * F)
    colf = [jnp.full((16,), f, jnp.int32) for f in range(F)]

    def chunk_body(ci, carry):
        p0 = ci * C
        pltpu.sync_copy(cx_h.at[pl.ds(base + p0, C)], cx)
        pltpu.sync_copy(cy_h.at[pl.ds(base + p0, C)], cy)
        pltpu.sync_copy(cz_h.at[pl.ds(base + p0, C)], cz)

        def phase_a(g, carry2):
            goff = g * 16
            px = cx[pl.ds(goff, 16)]
            py = cy[pl.ds(goff, 16)]
            pz = cz[pl.ds(goff, 16)]
            for l in range(NLVL):
                scale = jnp.float32(2.0 ** l * 16.0 - 1.0)
                posx = px * scale + 0.5
                posy = py * scale + 0.5
                posz = pz * scale + 0.5
                ix = posx.astype(jnp.int32)
                iy = posy.astype(jnp.int32)
                iz = posz.astype(jnp.int32)
                fx = posx - ix.astype(jnp.float32)
                fy = posy - iy.astype(jnp.float32)
                fz = posz - iz.astype(jnp.float32)
                u = [1.0 - fx, fx]
                v = [1.0 - fy, fy]
                t = [1.0 - fz, fz]
                vt = [[v[0] * t[0], v[0] * t[1]], [v[1] * t[0], v[1] * t[1]]]
                if l < 3:
                    s1 = l + 4
                    mask = (1 << (3 * l + 12)) - 1
                    a = [ix, ix + 1]
                    b0 = iy << s1
                    b = [b0, b0 + (1 << s1)]
                    c0 = iz << (2 * s1)
                    c = [c0, c0 + (1 << (2 * s1))]
                    for k in range(8):
                        bx, by, bz = (k >> 2) & 1, (k >> 1) & 1, k & 1
                        h = (a[bx] + b[by] + c[bz]) & mask
                        idx_b[pl.ds((l * 8 + k) * C + goff, 16)] = h
                        wgt_b[pl.ds((l * 8 + k) * C + goff, 16)] = u[bx] * vt[by][bz]
                else:
                    a = [ix, ix + 1]
                    b0 = iy * _P1
                    b = [b0, b0 + _P1]
                    c0 = iz * _P2
                    c = [c0, c0 + _P2]
                    for k in range(8):
                        bx, by, bz = (k >> 2) & 1, (k >> 1) & 1, k & 1
                        h = (a[bx] ^ b[by] ^ c[bz]) & _XMASK
                        idx_b[pl.ds((l * 8 + k) * C + goff, 16)] = h
                        wgt_b[pl.ds((l * 8 + k) * C + goff, 16)] = u[bx] * vt[by][bz]
            return carry2

        lax.fori_loop(0, G, phase_a, 0)

        pltpu.async_copy(table_h.at[idx_b], gath_b, sem).wait()

        def phase_c(g, carry2):
            goff = g * 16
            for l in range(NLVL):
                acc = [None] * F
                for k in range(8):
                    rb = (l * 8 + k) * C + goff
                    wk = wgt_b[pl.ds(rb, 16)]
                    rows = iota + rb
                    for f in range(F):
                        r = plsc.load_gather(gath_b, [rows, colf[f]])
                        term = wk * r
                        acc[f] = term if k == 0 else acc[f] + term
                ob = goff * (NLVL * F) + l * F
                for f in range(F):
                    plsc.store_scatter(out_b, [iota_out + (ob + f)], acc[f])
            return carry2

        lax.fori_loop(0, G, phase_c, 0)

        pltpu.sync_copy(out_b,
                        out_h.at[pl.ds((base + p0) * (NLVL * F), C * NLVL * F)])
        return carry

    lax.fori_loop(0, NCHUNK, chunk_body, 0)


def kernel(coords, params):
    coords = coords.astype(jnp.float32)
    cx = coords[:, 0]
    cy = coords[:, 1]
    cz = coords[:, 2]
    used = params[: R_USED * F].reshape(R_USED, F)
    table = jnp.zeros((R_USED, D), jnp.float32).at[:, :F].set(used)
    flat = _hashgrid_sc(cx, cy, cz, table)
    return flat.reshape(N_POINTS, NLVL * F)


# speedup vs baseline: 11.2855x; 1.4992x over previous
import functools

import jax
import jax.numpy as jnp
from jax import lax
from jax.experimental import pallas as pl
from jax.experimental.pallas import tpu as pltpu
from jax.experimental.pallas import tpu_sc as plsc

N_POINTS = 131072
NLVL = 16
F = 4
D = 8
R_USED = 1 << 19
NW = 32
PW = N_POINTS // NW
C = 32
G = C // 16
NCHUNK = PW // C
B = NLVL * 8 * C

_P1 = -1640531535
_P2 = 805459861
_XMASK = (1 << 19) - 1

_mesh = plsc.VectorSubcoreMesh(core_axis_name="c", subcore_axis_name="s")


@functools.partial(
    pl.kernel,
    mesh=_mesh,
    out_type=jax.ShapeDtypeStruct((N_POINTS * NLVL * F,), jnp.float32),
    scratch_types=[
        pltpu.VMEM((PW,), jnp.float32),
        pltpu.VMEM((PW,), jnp.float32),
        pltpu.VMEM((PW,), jnp.float32),
        pltpu.VMEM((B,), jnp.int32),
        pltpu.VMEM((B,), jnp.int32),
        pltpu.VMEM((B,), jnp.float32),
        pltpu.VMEM((B,), jnp.float32),
        pltpu.VMEM((B, D), jnp.float32),
        pltpu.VMEM((B, D), jnp.float32),
        pltpu.VMEM((C * NLVL * F,), jnp.float32),
        pltpu.SemaphoreType.DMA,
        pltpu.SemaphoreType.DMA,
    ],
    compiler_params=pltpu.CompilerParams(
        needs_layout_passes=False, use_tc_tiling_on_sc=False),
)
def _hashgrid_sc(cx_h, cy_h, cz_h, table_h, out_h,
                 cx, cy, cz, idx0, idx1, wgt0, wgt1, g0, g1, out_b,
                 sem0, sem1):
    wid = lax.axis_index("s") * 2 + lax.axis_index("c")
    base = wid * PW
    pltpu.sync_copy(cx_h.at[pl.ds(base, PW)], cx)
    pltpu.sync_copy(cy_h.at[pl.ds(base, PW)], cy)
    pltpu.sync_copy(cz_h.at[pl.ds(base, PW)], cz)

    idxs = (idx0, idx1)
    wgts = (wgt0, wgt1)
    gaths = (g0, g1)
    sems = (sem0, sem1)

    iota = lax.iota(jnp.int32, 16)
    iota_out = iota * (NLVL * F)
    colf = [jnp.full((16,), f, jnp.int32) for f in range(F)]

    def phase_a(ci, b):
        p0 = ci * C
        idx_b, wgt_b = idxs[b], wgts[b]

        def body(g, carry):
            goff = g * 16
            px = cx[pl.ds(p0 + goff, 16)]
            py = cy[pl.ds(p0 + goff, 16)]
            pz = cz[pl.ds(p0 + goff, 16)]
            for l in range(NLVL):
                scale = jnp.float32(2.0 ** l * 16.0 - 1.0)
                posx = px * scale + 0.5
                posy = py * scale + 0.5
                posz = pz * scale + 0.5
                ix = posx.astype(jnp.int32)
                iy = posy.astype(jnp.int32)
                iz = posz.astype(jnp.int32)
                fx = posx - ix.astype(jnp.float32)
                fy = posy - iy.astype(jnp.float32)
                fz = posz - iz.astype(jnp.float32)
                u = [1.0 - fx, fx]
                v = [1.0 - fy, fy]
                t = [1.0 - fz, fz]
                vt = [[v[0] * t[0], v[0] * t[1]], [v[1] * t[0], v[1] * t[1]]]
                if l < 3:
                    s1 = l + 4
                    mask = (1 << (3 * l + 12)) - 1
                    a = [ix, ix + 1]
                    b0v = iy << s1
                    bv = [b0v, b0v + (1 << s1)]
                    c0v = iz << (2 * s1)
                    cv = [c0v, c0v + (1 << (2 * s1))]
                    for k in range(8):
                        bx, by, bz = (k >> 2) & 1, (k >> 1) & 1, k & 1
                        h = (a[bx] + bv[by] + cv[bz]) & mask
                        idx_b[pl.ds((l * 8 + k) * C + goff, 16)] = h
                        wgt_b[pl.ds((l * 8 + k) * C + goff, 16)] = u[bx] * vt[by][bz]
                else:
                    a = [ix, ix + 1]
                    b0v = iy * _P1
                    bv = [b0v, b0v + _P1]
                    c0v = iz * _P2
                    cv = [c0v, c0v + _P2]
                    for k in range(8):
                        bx, by, bz = (k >> 2) & 1, (k >> 1) & 1, k & 1
                        h = (a[bx] ^ bv[by] ^ cv[bz]) & _XMASK
                        idx_b[pl.ds((l * 8 + k) * C + goff, 16)] = h
                        wgt_b[pl.ds((l * 8 + k) * C + goff, 16)] = u[bx] * vt[by][bz]
            return carry

        lax.fori_loop(0, G, body, 0)

    def fire(b):
        pltpu.async_copy(table_h.at[idxs[b]], gaths[b], sems[b])

    def wait(b):
        pltpu.make_async_copy(table_h.at[idxs[b]], gaths[b], sems[b]).wait()

    def combine(ci, b):
        gath_b, wgt_b = gaths[b], wgts[b]

        def body(g, carry):
            goff = g * 16
            for l in range(NLVL):
                acc = [None] * F
                for k in range(8):
                    rb = (l * 8 + k) * C + goff
                    wk = wgt_b[pl.ds(rb, 16)]
                    rows = iota + rb
                    for f in range(F):
                        r = plsc.load_gather(gath_b, [rows, colf[f]])
                        term = wk * r
                        acc[f] = term if k == 0 else acc[f] + term
                ob = goff * (NLVL * F) + l * F
                for f in range(F):
                    plsc.store_scatter(out_b, [iota_out + (ob + f)], acc[f])
            return carry

        lax.fori_loop(0, G, body, 0)
        pltpu.sync_copy(out_b,
                        out_h.at[pl.ds((base + ci * C) * (NLVL * F),
                                       C * NLVL * F)])

    phase_a(0, 0)
    fire(0)

    def outer(oi, carry):
        ci0 = oi * 2
        for b in range(2):
            ci = ci0 + b
            nxt = ci + 1

            @pl.when(nxt < NCHUNK)
            def _():
                phase_a(nxt, 1 - b)
                fire(1 - b)

            wait(b)
            combine(ci, b)
        return carry

    lax.fori_loop(0, NCHUNK // 2, outer, 0)


def kernel(coords, params):
    coords = coords.astype(jnp.float32)
    cx = coords[:, 0]
    cy = coords[:, 1]
    cz = coords[:, 2]
    used = params[: R_USED * F].reshape(R_USED, F)
    table = jnp.zeros((R_USED, D), jnp.float32).at[:, :F].set(used)
    flat = _hashgrid_sc(cx, cy, cz, table)
    return flat.reshape(N_POINTS, NLVL * F)


# speedup vs baseline: 15.0081x; 1.3299x over previous
import functools

import jax
import jax.numpy as jnp
from jax import lax
from jax.experimental import pallas as pl
from jax.experimental.pallas import tpu as pltpu
from jax.experimental.pallas import tpu_sc as plsc

N_POINTS = 131072
NLVL = 16
F = 4
D = 8
R_USED = 1 << 19
NW = 32
PW = N_POINTS // NW
C = 32
G = C // 16
NCHUNK = PW // C
B = NLVL * 8 * C

_P1 = -1640531535
_P2 = 805459861
_XMASK = (1 << 19) - 1

_mesh = plsc.VectorSubcoreMesh(core_axis_name="c", subcore_axis_name="s")


@functools.partial(
    pl.kernel,
    mesh=_mesh,
    out_type=jax.ShapeDtypeStruct((N_POINTS * NLVL * F,), jnp.float32),
    scratch_types=[
        pltpu.VMEM((PW,), jnp.float32),
        pltpu.VMEM((PW,), jnp.float32),
        pltpu.VMEM((PW,), jnp.float32),
        pltpu.VMEM((B,), jnp.int32),
        pltpu.VMEM((B,), jnp.int32),
        pltpu.VMEM((B,), jnp.float32),
        pltpu.VMEM((B,), jnp.float32),
        pltpu.VMEM((B,), jnp.int32),
        pltpu.VMEM((B,), jnp.int32),
        pltpu.VMEM((B, D), jnp.float32),
        pltpu.VMEM((B, D), jnp.float32),
        pltpu.VMEM((C * NLVL * F,), jnp.float32),
        pltpu.SemaphoreType.DMA,
        pltpu.SemaphoreType.DMA,
    ],
    compiler_params=pltpu.CompilerParams(
        needs_layout_passes=False, use_tc_tiling_on_sc=False),
)
def _hashgrid_sc(cx_h, cy_h, cz_h, table_h, out_h,
                 cx, cy, cz, idx0, idx1, wgt0, wgt1, par0, par1, g0, g1,
                 out_b, sem0, sem1):
    wid = lax.axis_index("s") * 2 + lax.axis_index("c")
    base = wid * PW
    pltpu.sync_copy(cx_h.at[pl.ds(base, PW)], cx)
    pltpu.sync_copy(cy_h.at[pl.ds(base, PW)], cy)
    pltpu.sync_copy(cz_h.at[pl.ds(base, PW)], cz)

    idxs = (idx0, idx1)
    wgts = (wgt0, wgt1)
    pars = (par0, par1)
    gaths = (g0, g1)
    sems = (sem0, sem1)

    iota = lax.iota(jnp.int32, 16)
    iota_out = iota * (NLVL * F)

    def phase_a(ci, b):
        p0 = ci * C
        idx_b, wgt_b, par_b = idxs[b], wgts[b], pars[b]

        def body(g, carry):
            goff = g * 16
            px = cx[pl.ds(p0 + goff, 16)]
            py = cy[pl.ds(p0 + goff, 16)]
            pz = cz[pl.ds(p0 + goff, 16)]
            for l in range(NLVL):
                scale = jnp.float32(2.0 ** l * 16.0 - 1.0)
                posx = px * scale + 0.5
                posy = py * scale + 0.5
                posz = pz * scale + 0.5
                ix = posx.astype(jnp.int32)
                iy = posy.astype(jnp.int32)
                iz = posz.astype(jnp.int32)
                fx = posx - ix.astype(jnp.float32)
                fy = posy - iy.astype(jnp.float32)
                fz = posz - iz.astype(jnp.float32)
                u = [1.0 - fx, fx]
                v = [1.0 - fy, fy]
                t = [1.0 - fz, fz]
                vt = [[v[0] * t[0], v[0] * t[1]], [v[1] * t[0], v[1] * t[1]]]
                if l < 3:
                    s1 = l + 4
                    mask = (1 << (3 * l + 12)) - 1
                    a = [ix, ix + 1]
                    b0v = iy << s1
                    bv = [b0v, b0v + (1 << s1)]
                    c0v = iz << (2 * s1)
                    cv = [c0v, c0v + (1 << (2 * s1))]
                    for k in range(8):
                        bx, by, bz = (k >> 2) & 1, (k >> 1) & 1, k & 1
                        h = (a[bx] + bv[by] + cv[bz]) & mask
                        idx_b[pl.ds((l * 8 + k) * C + goff, 16)] = h >> 1
                        par_b[pl.ds((l * 8 + k) * C + goff, 16)] = (h & 1) << 2
                        wgt_b[pl.ds((l * 8 + k) * C + goff, 16)] = u[bx] * vt[by][bz]
                else:
                    a = [ix, ix + 1]
                    b0v = iy * _P1
                    bv = [b0v, b0v + _P1]
                    c0v = iz * _P2
                    cv = [c0v, c0v + _P2]
                    for k in range(8):
                        bx, by, bz = (k >> 2) & 1, (k >> 1) & 1, k & 1
                        h = (a[bx] ^ bv[by] ^ cv[bz]) & _XMASK
                        idx_b[pl.ds((l * 8 + k) * C + goff, 16)] = h >> 1
                        par_b[pl.ds((l * 8 + k) * C + goff, 16)] = (h & 1) << 2
                        wgt_b[pl.ds((l * 8 + k) * C + goff, 16)] = u[bx] * vt[by][bz]
            return carry

        lax.fori_loop(0, G, body, 0)

    def fire(b):
        pltpu.async_copy(table_h.at[idxs[b]], gaths[b], sems[b])

    def wait(b):
        pltpu.make_async_copy(table_h.at[idxs[b]], gaths[b], sems[b]).wait()

    def combine(ci, b):
        gath_b, wgt_b, par_b = gaths[b], wgts[b], pars[b]

        def body(g, carry):
            goff = g * 16
            for l in range(NLVL):
                acc = [None] * F
                for k in range(8):
                    rb = (l * 8 + k) * C + goff
                    wk = wgt_b[pl.ds(rb, 16)]
                    pk = par_b[pl.ds(rb, 16)]
                    rows = iota + rb
                    for f in range(F):
                        r = plsc.load_gather(gath_b, [rows, pk + f])
                        term = wk * r
                        acc[f] = term if k == 0 else acc[f] + term
                ob = goff * (NLVL * F) + l * F
                for f in range(F):
                    plsc.store_scatter(out_b, [iota_out + (ob + f)], acc[f])
            return carry

        lax.fori_loop(0, G, body, 0)
        pltpu.sync_copy(out_b,
                        out_h.at[pl.ds((base + ci * C) * (NLVL * F),
                                       C * NLVL * F)])

    phase_a(0, 0)
    fire(0)

    def outer(oi, carry):
        ci0 = oi * 2
        for b in range(2):
            ci = ci0 + b
            nxt = ci + 1

            @pl.when(nxt < NCHUNK)
            def _():
                phase_a(nxt, 1 - b)
                fire(1 - b)

            wait(b)
            combine(ci, b)
        return carry

    lax.fori_loop(0, NCHUNK // 2, outer, 0)


def kernel(coords, params):
    coords = coords.astype(jnp.float32)
    cx = coords[:, 0]
    cy = coords[:, 1]
    cz = coords[:, 2]
    table = params.reshape(-1, D)
    flat = _hashgrid_sc(cx, cy, cz, table)
    return flat.reshape(N_POINTS, NLVL * F)


# speedup vs baseline: 15.2869x; 1.0186x over previous
import functools

import jax
import jax.numpy as jnp
from jax import lax
from jax.experimental import pallas as pl
from jax.experimental.pallas import tpu as pltpu
from jax.experimental.pallas import tpu_sc as plsc

N_POINTS = 131072
NLVL = 16
F = 4
D = 8
R_USED = 1 << 19
NW = 32
PW = N_POINTS // NW
C = 32
G = C // 16
NCHUNK = PW // C
B = NLVL * 8 * C

_P1 = -1640531535
_P2 = 805459861
_XMASK = (1 << 19) - 1

_mesh = plsc.VectorSubcoreMesh(core_axis_name="c", subcore_axis_name="s")


@functools.partial(
    pl.kernel,
    mesh=_mesh,
    out_type=jax.ShapeDtypeStruct((N_POINTS * NLVL * F,), jnp.float32),
    scratch_types=[
        pltpu.VMEM((PW,), jnp.float32),
        pltpu.VMEM((PW,), jnp.float32),
        pltpu.VMEM((PW,), jnp.float32),
        pltpu.VMEM((B,), jnp.int32),
        pltpu.VMEM((B,), jnp.int32),
        pltpu.VMEM((B,), jnp.float32),
        pltpu.VMEM((B,), jnp.float32),
        pltpu.VMEM((B,), jnp.int32),
        pltpu.VMEM((B,), jnp.int32),
        pltpu.VMEM((B, D), jnp.float32),
        pltpu.VMEM((B, D), jnp.float32),
        pltpu.VMEM((C * NLVL * F,), jnp.float32),
        pltpu.VMEM((C * NLVL * F,), jnp.float32),
        pltpu.SemaphoreType.DMA,
        pltpu.SemaphoreType.DMA,
        pltpu.SemaphoreType.DMA,
        pltpu.SemaphoreType.DMA,
    ],
    compiler_params=pltpu.CompilerParams(
        needs_layout_passes=False, use_tc_tiling_on_sc=False),
)
def _hashgrid_sc(cx_h, cy_h, cz_h, table_h, out_h,
                 cx, cy, cz, idx0, idx1, wgt0, wgt1, par0, par1, g0, g1,
                 out0, out1, sem0, sem1, osem0, osem1):
    wid = lax.axis_index("s") * 2 + lax.axis_index("c")
    base = wid * PW
    pltpu.sync_copy(cx_h.at[pl.ds(base, PW)], cx)
    pltpu.sync_copy(cy_h.at[pl.ds(base, PW)], cy)
    pltpu.sync_copy(cz_h.at[pl.ds(base, PW)], cz)

    idxs = (idx0, idx1)
    wgts = (wgt0, wgt1)
    pars = (par0, par1)
    gaths = (g0, g1)
    sems = (sem0, sem1)
    outs = (out0, out1)
    osems = (osem0, osem1)

    iota = lax.iota(jnp.int32, 16)
    iota_out = iota * (NLVL * F)

    def phase_a(ci, b):
        p0 = ci * C
        idx_b, wgt_b, par_b = idxs[b], wgts[b], pars[b]

        def body(g, carry):
            goff = g * 16
            px = cx[pl.ds(p0 + goff, 16)]
            py = cy[pl.ds(p0 + goff, 16)]
            pz = cz[pl.ds(p0 + goff, 16)]
            for l in range(NLVL):
                scale = jnp.float32(2.0 ** l * 16.0 - 1.0)
                posx = px * scale + 0.5
                posy = py * scale + 0.5
                posz = pz * scale + 0.5
                ix = posx.astype(jnp.int32)
                iy = posy.astype(jnp.int32)
                iz = posz.astype(jnp.int32)
                fx = posx - ix.astype(jnp.float32)
                fy = posy - iy.astype(jnp.float32)
                fz = posz - iz.astype(jnp.float32)
                u = [1.0 - fx, fx]
                v = [1.0 - fy, fy]
                t = [1.0 - fz, fz]
                vt = [[v[0] * t[0], v[0] * t[1]], [v[1] * t[0], v[1] * t[1]]]
                if l < 3:
                    s1 = l + 4
                    mask = (1 << (3 * l + 12)) - 1
                    a = [ix, ix + 1]
                    b0v = iy << s1
                    bv = [b0v, b0v + (1 << s1)]
                    c0v = iz << (2 * s1)
                    cv = [c0v, c0v + (1 << (2 * s1))]
                    for k in range(8):
                        bx, by, bz = (k >> 2) & 1, (k >> 1) & 1, k & 1
                        h = (a[bx] + bv[by] + cv[bz]) & mask
                        idx_b[pl.ds((l * 8 + k) * C + goff, 16)] = h >> 1
                        par_b[pl.ds((l * 8 + k) * C + goff, 16)] = (h & 1) << 2
                        wgt_b[pl.ds((l * 8 + k) * C + goff, 16)] = u[bx] * vt[by][bz]
                else:
                    a = [ix, ix + 1]
                    b0v = iy * _P1
                    bv = [b0v, b0v + _P1]
                    c0v = iz * _P2
                    cv = [c0v, c0v + _P2]
                    for k in range(8):
                        bx, by, bz = (k >> 2) & 1, (k >> 1) & 1, k & 1
                        h = (a[bx] ^ bv[by] ^ cv[bz]) & _XMASK
                        idx_b[pl.ds((l * 8 + k) * C + goff, 16)] = h >> 1
                        par_b[pl.ds((l * 8 + k) * C + goff, 16)] = (h & 1) << 2
                        wgt_b[pl.ds((l * 8 + k) * C + goff, 16)] = u[bx] * vt[by][bz]
            return carry

        lax.fori_loop(0, G, body, 0)

    def fire(b):
        pltpu.async_copy(table_h.at[idxs[b]], gaths[b], sems[b])

    def wait(b):
        pltpu.make_async_copy(table_h.at[idxs[b]], gaths[b], sems[b]).wait()

    def out_wait(b):
        pltpu.make_async_copy(
            outs[b], out_h.at[pl.ds(base * (NLVL * F), C * NLVL * F)],
            osems[b]).wait()

    def combine(ci, b):
        gath_b, wgt_b, par_b = gaths[b], wgts[b], pars[b]
        out_b = outs[b]

        @pl.when(ci >= 2)
        def _():
            out_wait(b)

        def body(g, carry):
            goff = g * 16
            for l in range(NLVL):
                acc = [None] * F
                for k in range(8):
                    rb = (l * 8 + k) * C + goff
                    wk = wgt_b[pl.ds(rb, 16)]
                    pk = par_b[pl.ds(rb, 16)]
                    rows = iota + rb
                    for f in range(F):
                        r = plsc.load_gather(gath_b, [rows, pk + f])
                        term = wk * r
                        acc[f] = term if k == 0 else acc[f] + term
                ob = goff * (NLVL * F) + l * F
                for f in range(F):
                    plsc.store_scatter(out_b, [iota_out + (ob + f)], acc[f])
            return carry

        lax.fori_loop(0, G, body, 0)
        pltpu.async_copy(out_b,
                         out_h.at[pl.ds((base + ci * C) * (NLVL * F),
                                        C * NLVL * F)],
                         osems[b])

    phase_a(0, 0)
    fire(0)

    def outer(oi, carry):
        ci0 = oi * 2
        for b in range(2):
            ci = ci0 + b
            nxt = ci + 1

            @pl.when(nxt < NCHUNK)
            def _():
                phase_a(nxt, 1 - b)
                fire(1 - b)

            wait(b)
            combine(ci, b)
        return carry

    lax.fori_loop(0, NCHUNK // 2, outer, 0)
    out_wait(0)
    out_wait(1)


def kernel(coords, params):
    coords = coords.astype(jnp.float32)
    cx = coords[:, 0]
    cy = coords[:, 1]
    cz = coords[:, 2]
    table = params.reshape(-1, D)
    flat = _hashgrid_sc(cx, cy, cz, table)
    return flat.reshape(N_POINTS, NLVL * F)


# speedup vs baseline: 15.3864x; 1.0065x over previous
import functools

import jax
import jax.numpy as jnp
from jax import lax
from jax.experimental import pallas as pl
from jax.experimental.pallas import tpu as pltpu
from jax.experimental.pallas import tpu_sc as plsc

N_POINTS = 131072
NLVL = 16
F = 4
D = 8
NW = 32
PW = N_POINTS // NW
C = 32
G = C // 16
NCHUNK = PW // C
NREG = NLVL * 8
BD = (NLVL - 1) * 8 * C
L0ROWS = 4096 // 2

_P1 = -1640531535
_P2 = 805459861
_XMASK = (1 << 19) - 1

_mesh = plsc.VectorSubcoreMesh(core_axis_name="c", subcore_axis_name="s")


@functools.partial(
    pl.kernel,
    mesh=_mesh,
    out_type=jax.ShapeDtypeStruct((N_POINTS * NLVL * F,), jnp.float32),
    scratch_types=[
        pltpu.VMEM((PW,), jnp.float32),
        pltpu.VMEM((PW,), jnp.float32),
        pltpu.VMEM((PW,), jnp.float32),
        pltpu.VMEM((L0ROWS, D), jnp.float32),
        pltpu.VMEM((BD,), jnp.int32),
        pltpu.VMEM((BD,), jnp.int32),
        pltpu.VMEM((8 * C,), jnp.int32),
        pltpu.VMEM((8 * C,), jnp.int32),
        pltpu.VMEM((NREG * C,), jnp.float32),
        pltpu.VMEM((NREG * C,), jnp.float32),
        pltpu.VMEM((NREG * C,), jnp.int32),
        pltpu.VMEM((NREG * C,), jnp.int32),
        pltpu.VMEM((BD, D), jnp.float32),
        pltpu.VMEM((BD, D), jnp.float32),
        pltpu.VMEM((C * NLVL * F,), jnp.float32),
        pltpu.VMEM((C * NLVL * F,), jnp.float32),
        pltpu.SemaphoreType.DMA,
        pltpu.SemaphoreType.DMA,
        pltpu.SemaphoreType.DMA,
        pltpu.SemaphoreType.DMA,
    ],
    compiler_params=pltpu.CompilerParams(
        needs_layout_passes=False, use_tc_tiling_on_sc=False),
)
def _hashgrid_sc(cx_h, cy_h, cz_h, table_h, out_h,
                 cx, cy, cz, lvl0, idx0, idx1, lix0, lix1, wgt0, wgt1,
                 par0, par1, g0, g1, out0, out1, sem0, sem1, osem0, osem1):
    wid = lax.axis_index("s") * 2 + lax.axis_index("c")
    base = wid * PW
    pltpu.sync_copy(cx_h.at[pl.ds(base, PW)], cx)
    pltpu.sync_copy(cy_h.at[pl.ds(base, PW)], cy)
    pltpu.sync_copy(cz_h.at[pl.ds(base, PW)], cz)
    pltpu.sync_copy(table_h.at[pl.ds(0, L0ROWS)], lvl0)

    idxs = (idx0, idx1)
    lixs = (lix0, lix1)
    wgts = (wgt0, wgt1)
    pars = (par0, par1)
    gaths = (g0, g1)
    sems = (sem0, sem1)
    outs = (out0, out1)
    osems = (osem0, osem1)

    iota = lax.iota(jnp.int32, 16)
    iota_out = iota * (NLVL * F)

    def phase_a(ci, b):
        p0 = ci * C
        idx_b, lix_b, wgt_b, par_b = idxs[b], lixs[b], wgts[b], pars[b]

        def body(g, carry):
            goff = g * 16
            px = cx[pl.ds(p0 + goff, 16)]
            py = cy[pl.ds(p0 + goff, 16)]
            pz = cz[pl.ds(p0 + goff, 16)]
            for l in range(NLVL):
                scale = jnp.float32(2.0 ** l * 16.0 - 1.0)
                posx = px * scale + 0.5
                posy = py * scale + 0.5
                posz = pz * scale + 0.5
                ix = posx.astype(jnp.int32)
                iy = posy.astype(jnp.int32)
                iz = posz.astype(jnp.int32)
                fx = posx - ix.astype(jnp.float32)
                fy = posy - iy.astype(jnp.float32)
                fz = posz - iz.astype(jnp.float32)
                u = [1.0 - fx, fx]
                v = [1.0 - fy, fy]
                t = [1.0 - fz, fz]
                vt = [[v[0] * t[0], v[0] * t[1]], [v[1] * t[0], v[1] * t[1]]]
                if l < 3:
                    s1 = l + 4
                    mask = (1 << (3 * l + 12)) - 1
                    a = [ix, ix + 1]
                    b0v = iy << s1
                    bv = [b0v, b0v + (1 << s1)]
                    c0v = iz << (2 * s1)
                    cv = [c0v, c0v + (1 << (2 * s1))]
                    hash8 = [(a[(k >> 2) & 1] + bv[(k >> 1) & 1] + cv[k & 1])
                             & mask for k in range(8)]
                else:
                    a = [ix, ix + 1]
                    b0v = iy * _P1
                    bv = [b0v, b0v + _P1]
                    c0v = iz * _P2
                    cv = [c0v, c0v + _P2]
                    hash8 = [(a[(k >> 2) & 1] ^ bv[(k >> 1) & 1] ^ cv[k & 1])
                             & _XMASK for k in range(8)]
                for k in range(8):
                    h = hash8[k]
                    bx, by, bz = (k >> 2) & 1, (k >> 1) & 1, k & 1
                    reg = (l - 1) * 8 + k if l >= 1 else 120 + k
                    if l >= 1:
                        idx_b[pl.ds(reg * C + goff, 16)] = h >> 1
                    else:
                        lix_b[pl.ds(k * C + goff, 16)] = h >> 1
                    par_b[pl.ds(reg * C + goff, 16)] = (h & 1) << 2
                    wgt_b[pl.ds(reg * C + goff, 16)] = u[bx] * vt[by][bz]
            return carry

        lax.fori_loop(0, G, body, 0)

    def fire(b):
        pltpu.async_copy(table_h.at[idxs[b]], gaths[b], sems[b])

    def wait(b):
        pltpu.make_async_copy(table_h.at[idxs[b]], gaths[b], sems[b]).wait()

    def out_wait(b):
        pltpu.make_async_copy(
            outs[b], out_h.at[pl.ds(base * (NLVL * F), C * NLVL * F)],
            osems[b]).wait()

    def combine(ci, b):
        gath_b, lix_b, wgt_b, par_b = gaths[b], lixs[b], wgts[b], pars[b]
        out_b = outs[b]

        @pl.when(ci >= 2)
        def _():
            out_wait(b)

        def body(g, carry):
            goff = g * 16
            for l in range(NLVL):
                acc = [None] * F
                for k in range(8):
                    reg = (l - 1) * 8 + k if l >= 1 else 120 + k
                    rb = reg * C + goff
                    wk = wgt_b[pl.ds(rb, 16)]
                    pk = par_b[pl.ds(rb, 16)]
                    if l >= 1:
                        src = gath_b
                        rows = iota + rb
                    else:
                        src = lvl0
                        rows = lix_b[pl.ds(k * C + goff, 16)]
                    for f in range(F):
                        r = plsc.load_gather(src, [rows, pk + f])
                        term = wk * r
                        acc[f] = term if k == 0 else acc[f] + term
                ob = goff * (NLVL * F) + l * F
                for f in range(F):
                    plsc.store_scatter(out_b, [iota_out + (ob + f)], acc[f])
            return carry

        lax.fori_loop(0, G, body, 0)
        pltpu.async_copy(out_b,
                         out_h.at[pl.ds((base + ci * C) * (NLVL * F),
                                        C * NLVL * F)],
                         osems[b])

    phase_a(0, 0)
    fire(0)

    def outer(oi, carry):
        ci0 = oi * 2
        for b in range(2):
            ci = ci0 + b
            nxt = ci + 1

            @pl.when(nxt < NCHUNK)
            def _():
                phase_a(nxt, 1 - b)
                fire(1 - b)

            wait(b)
            combine(ci, b)
        return carry

    lax.fori_loop(0, NCHUNK // 2, outer, 0)
    out_wait(0)
    out_wait(1)


def kernel(coords, params):
    coords = coords.astype(jnp.float32)
    cx = coords[:, 0]
    cy = coords[:, 1]
    cz = coords[:, 2]
    table = params.reshape(-1, D)
    flat = _hashgrid_sc(cx, cy, cz, table)
    return flat.reshape(N_POINTS, NLVL * F)


# speedup vs baseline: 22.0161x; 1.4309x over previous
import functools

import jax
import jax.numpy as jnp
from jax import lax
from jax.experimental import pallas as pl
from jax.experimental.pallas import tpu as pltpu
from jax.experimental.pallas import tpu_sc as plsc

N_POINTS = 131072
NLVL = 16
F = 4
D = 8
NW = 32
PW = N_POINTS // NW
C = 32
G = C // 16
NCHUNK = PW // C
NREG = NLVL * 8
BD = (NLVL - 1) * 8 * C
L0ROWS = 4096 // 2

_P1 = -1640531535
_P2 = 805459861
_XMASK = (1 << 19) - 1

_mesh = plsc.VectorSubcoreMesh(core_axis_name="c", subcore_axis_name="s")


@functools.partial(
    pl.kernel,
    mesh=_mesh,
    out_type=jax.ShapeDtypeStruct((N_POINTS * NLVL * F,), jnp.float32),
    scratch_types=[
        pltpu.VMEM((PW,), jnp.float32),
        pltpu.VMEM((PW,), jnp.float32),
        pltpu.VMEM((PW,), jnp.float32),
        pltpu.VMEM((L0ROWS, D), jnp.float32),
        pltpu.VMEM((BD,), jnp.int32),
        pltpu.VMEM((BD,), jnp.int32),
        pltpu.VMEM((8 * C,), jnp.int32),
        pltpu.VMEM((8 * C,), jnp.int32),
        pltpu.VMEM((NREG * C,), jnp.float32),
        pltpu.VMEM((NREG * C,), jnp.float32),
        pltpu.VMEM((NREG * C,), jnp.int32),
        pltpu.VMEM((NREG * C,), jnp.int32),
        pltpu.VMEM((BD, D), jnp.float32),
        pltpu.VMEM((BD, D), jnp.float32),
        pltpu.VMEM((C * NLVL * F,), jnp.float32),
        pltpu.VMEM((C * NLVL * F,), jnp.float32),
        pltpu.SemaphoreType.DMA,
        pltpu.SemaphoreType.DMA,
        pltpu.SemaphoreType.DMA,
        pltpu.SemaphoreType.DMA,
    ],
    compiler_params=pltpu.CompilerParams(
        needs_layout_passes=False, use_tc_tiling_on_sc=False),
)
def _hashgrid_sc(cx_h, cy_h, cz_h, table_h, out_h,
                 cx, cy, cz, lvl0, idx0, idx1, lix0, lix1, wgt0, wgt1,
                 par0, par1, g0, g1, out0, out1, sem0, sem1, osem0, osem1):
    wid = lax.axis_index("s") * 2 + lax.axis_index("c")
    base = wid * PW
    pltpu.sync_copy(cx_h.at[pl.ds(base, PW)], cx)
    pltpu.sync_copy(cy_h.at[pl.ds(base, PW)], cy)
    pltpu.sync_copy(cz_h.at[pl.ds(base, PW)], cz)
    pltpu.sync_copy(table_h.at[pl.ds(0, L0ROWS)], lvl0)

    idxs = (idx0, idx1)
    lixs = (lix0, lix1)
    wgts = (wgt0, wgt1)
    pars = (par0, par1)
    gaths = (g0, g1)
    sems = (sem0, sem1)
    outs = (out0, out1)
    osems = (osem0, osem1)

    iota = lax.iota(jnp.int32, 16)
    iota_out = iota * (NLVL * F)

    def phase_a(ci, b):
        p0 = ci * C
        idx_b, lix_b, wgt_b, par_b = idxs[b], lixs[b], wgts[b], pars[b]

        def body(g, carry):
            goff = g * 16
            px = cx[pl.ds(p0 + goff, 16)]
            py = cy[pl.ds(p0 + goff, 16)]
            pz = cz[pl.ds(p0 + goff, 16)]

            def frac_weights(posx, posy, posz):
                ix = posx.astype(jnp.int32)
                iy = posy.astype(jnp.int32)
                iz = posz.astype(jnp.int32)
                fx = posx - ix.astype(jnp.float32)
                fy = posy - iy.astype(jnp.float32)
                fz = posz - iz.astype(jnp.float32)
                u = [1.0 - fx, fx]
                v = [1.0 - fy, fy]
                t = [1.0 - fz, fz]
                vt = [[v[0] * t[0], v[0] * t[1]], [v[1] * t[0], v[1] * t[1]]]
                w8 = [u[(k >> 2) & 1] * vt[(k >> 1) & 1][k & 1]
                      for k in range(8)]
                return ix, iy, iz, w8

            for l in range(3):
                scale = jnp.float32(2.0 ** l * 16.0 - 1.0)
                ix, iy, iz, w8 = frac_weights(px * scale + 0.5,
                                              py * scale + 0.5,
                                              pz * scale + 0.5)
                s1 = l + 4
                mask = (1 << (3 * l + 12)) - 1
                a = [ix, ix + 1]
                b0v = iy << s1
                bv = [b0v, b0v + (1 << s1)]
                c0v = iz << (2 * s1)
                cv = [c0v, c0v + (1 << (2 * s1))]
                for k in range(8):
                    h = (a[(k >> 2) & 1] + bv[(k >> 1) & 1] + cv[k & 1]) & mask
                    reg = (l - 1) * 8 + k if l >= 1 else 120 + k
                    if l >= 1:
                        idx_b[pl.ds(reg * C + goff, 16)] = h >> 1
                    else:
                        lix_b[pl.ds(k * C + goff, 16)] = h >> 1
                    par_b[pl.ds(reg * C + goff, 16)] = (h & 1) << 2
                    wgt_b[pl.ds(reg * C + goff, 16)] = w8[k]

            def lbody(li, carry2):
                l = li + 3
                scale = lax.bitcast_convert_type(
                    (l + 131) << 23, jnp.float32) - 1.0
                ix, iy, iz, w8 = frac_weights(px * scale + 0.5,
                                              py * scale + 0.5,
                                              pz * scale + 0.5)
                a = [ix, ix + 1]
                b0v = iy * _P1
                bv = [b0v, b0v + _P1]
                c0v = iz * _P2
                cv = [c0v, c0v + _P2]
                rbase = ((li + 2) * 8) * C + goff
                for k in range(8):
                    h = (a[(k >> 2) & 1] ^ bv[(k >> 1) & 1] ^ cv[k & 1]) \
                        & _XMASK
                    idx_b[pl.ds(rbase + k * C, 16)] = h >> 1
                    par_b[pl.ds(rbase + k * C, 16)] = (h & 1) << 2
                    wgt_b[pl.ds(rbase + k * C, 16)] = w8[k]
                return carry2

            lax.fori_loop(0, 13, lbody, 0)
            return carry

        lax.fori_loop(0, G, body, 0)

    def fire(b):
        pltpu.async_copy(table_h.at[idxs[b]], gaths[b], sems[b])

    def wait(b):
        pltpu.make_async_copy(table_h.at[idxs[b]], gaths[b], sems[b]).wait()

    def out_wait(b):
        pltpu.make_async_copy(
            outs[b], out_h.at[pl.ds(base * (NLVL * F), C * NLVL * F)],
            osems[b]).wait()

    def combine(ci, b):
        gath_b, lix_b, wgt_b, par_b = gaths[b], lixs[b], wgts[b], pars[b]
        out_b = outs[b]

        @pl.when(ci >= 2)
        def _():
            out_wait(b)

        def body(g, carry):
            goff = g * 16
            obase = goff * (NLVL * F)

            acc = [None] * F
            for k in range(8):
                rb = (120 + k) * C + goff
                wk = wgt_b[pl.ds(rb, 16)]
                pk = par_b[pl.ds(rb, 16)]
                rows = lix_b[pl.ds(k * C + goff, 16)]
                for f in range(F):
                    r = plsc.load_gather(lvl0, [rows, pk + f])
                    term = wk * r
                    acc[f] = term if k == 0 else acc[f] + term
            for f in range(F):
                plsc.store_scatter(out_b, [iota_out + (obase + f)], acc[f])

            def lbody(li, carry2):
                rb0 = (li * 8) * C + goff
                acc = [None] * F
                for k in range(8):
                    rb = rb0 + k * C
                    wk = wgt_b[pl.ds(rb, 16)]
                    pk = par_b[pl.ds(rb, 16)]
                    rows = iota + rb
                    for f in range(F):
                        r = plsc.load_gather(gath_b, [rows, pk + f])
                        term = wk * r
                        acc[f] = term if k == 0 else acc[f] + term
                ob = obase + (li + 1) * F
                for f in range(F):
                    plsc.store_scatter(out_b, [iota_out + (ob + f)], acc[f])
                return carry2

            lax.fori_loop(0, NLVL - 1, lbody, 0)
            return carry

        lax.fori_loop(0, G, body, 0)
        pltpu.async_copy(out_b,
                         out_h.at[pl.ds((base + ci * C) * (NLVL * F),
                                        C * NLVL * F)],
                         osems[b])

    phase_a(0, 0)
    fire(0)

    def outer(oi, carry):
        ci0 = oi * 2
        for b in range(2):
            ci = ci0 + b
            nxt = ci + 1

            @pl.when(nxt < NCHUNK)
            def _():
                phase_a(nxt, 1 - b)
                fire(1 - b)

            wait(b)
            combine(ci, b)
        return carry

    lax.fori_loop(0, NCHUNK // 2, outer, 0)
    out_wait(0)
    out_wait(1)


def kernel(coords, params):
    coords = coords.astype(jnp.float32)
    cx = coords[:, 0]
    cy = coords[:, 1]
    cz = coords[:, 2]
    table = params.reshape(-1, D)
    flat = _hashgrid_sc(cx, cy, cz, table)
    return flat.reshape(N_POINTS, NLVL * F)


# speedup vs baseline: 22.4032x; 1.0176x over previous
import functools

import jax
import jax.numpy as jnp
from jax import lax
from jax.experimental import pallas as pl
from jax.experimental.pallas import tpu as pltpu
from jax.experimental.pallas import tpu_sc as plsc

N_POINTS = 131072
NLVL = 16
F = 4
D = 8
NW = 32
PW = N_POINTS // NW
C = 32
G = C // 16
NCHUNK = PW // C
NREG = NLVL * 8
BD = (NLVL - 1) * 8 * C
L0ROWS = 4096 // 2

_P1 = -1640531535
_P2 = 805459861
_XMASK = (1 << 19) - 1

_mesh = plsc.VectorSubcoreMesh(core_axis_name="c", subcore_axis_name="s")


@functools.partial(
    pl.kernel,
    mesh=_mesh,
    out_type=jax.ShapeDtypeStruct((N_POINTS // 8, 8, 128), jnp.float32),
    scratch_types=[
        pltpu.VMEM((PW,), jnp.float32),
        pltpu.VMEM((PW,), jnp.float32),
        pltpu.VMEM((PW,), jnp.float32),
        pltpu.VMEM((L0ROWS, D), jnp.float32),
        pltpu.VMEM((BD,), jnp.int32),
        pltpu.VMEM((BD,), jnp.int32),
        pltpu.VMEM((8 * C,), jnp.int32),
        pltpu.VMEM((8 * C,), jnp.int32),
        pltpu.VMEM((NREG * C,), jnp.float32),
        pltpu.VMEM((NREG * C,), jnp.float32),
        pltpu.VMEM((NREG * C,), jnp.int32),
        pltpu.VMEM((NREG * C,), jnp.int32),
        pltpu.VMEM((BD, D), jnp.float32),
        pltpu.VMEM((BD, D), jnp.float32),
        pltpu.VMEM((C // 8, 8, NLVL * F), jnp.float32),
        pltpu.VMEM((C // 8, 8, NLVL * F), jnp.float32),
        pltpu.SemaphoreType.DMA,
        pltpu.SemaphoreType.DMA,
        pltpu.SemaphoreType.DMA,
        pltpu.SemaphoreType.DMA,
    ],
    compiler_params=pltpu.CompilerParams(
        needs_layout_passes=False, use_tc_tiling_on_sc=False),
)
def _hashgrid_sc(cx_h, cy_h, cz_h, table_h, out_h,
                 cx, cy, cz, lvl0, idx0, idx1, lix0, lix1, wgt0, wgt1,
                 par0, par1, g0, g1, out0, out1, sem0, sem1, osem0, osem1):
    wid = lax.axis_index("s") * 2 + lax.axis_index("c")
    base = wid * PW
    pltpu.sync_copy(cx_h.at[pl.ds(base, PW)], cx)
    pltpu.sync_copy(cy_h.at[pl.ds(base, PW)], cy)
    pltpu.sync_copy(cz_h.at[pl.ds(base, PW)], cz)
    pltpu.sync_copy(table_h.at[pl.ds(0, L0ROWS)], lvl0)

    idxs = (idx0, idx1)
    lixs = (lix0, lix1)
    wgts = (wgt0, wgt1)
    pars = (par0, par1)
    gaths = (g0, g1)
    sems = (sem0, sem1)
    outs = (out0, out1)
    osems = (osem0, osem1)

    iota = lax.iota(jnp.int32, 16)
    iota_t = iota >> 3
    iota_s = iota & 7

    def phase_a(ci, b):
        p0 = ci * C
        idx_b, lix_b, wgt_b, par_b = idxs[b], lixs[b], wgts[b], pars[b]

        def body(g, carry):
            goff = g * 16
            px = cx[pl.ds(p0 + goff, 16)]
            py = cy[pl.ds(p0 + goff, 16)]
            pz = cz[pl.ds(p0 + goff, 16)]

            def frac_weights(posx, posy, posz):
                ix = posx.astype(jnp.int32)
                iy = posy.astype(jnp.int32)
                iz = posz.astype(jnp.int32)
                fx = posx - ix.astype(jnp.float32)
                fy = posy - iy.astype(jnp.float32)
                fz = posz - iz.astype(jnp.float32)
                u = [1.0 - fx, fx]
                v = [1.0 - fy, fy]
                t = [1.0 - fz, fz]
                vt = [[v[0] * t[0], v[0] * t[1]], [v[1] * t[0], v[1] * t[1]]]
                w8 = [u[(k >> 2) & 1] * vt[(k >> 1) & 1][k & 1]
                      for k in range(8)]
                return ix, iy, iz, w8

            for l in range(3):
                scale = jnp.float32(2.0 ** l * 16.0 - 1.0)
                ix, iy, iz, w8 = frac_weights(px * scale + 0.5,
                                              py * scale + 0.5,
                                              pz * scale + 0.5)
                s1 = l + 4
                mask = (1 << (3 * l + 12)) - 1
                a = [ix, ix + 1]
                b0v = iy << s1
                bv = [b0v, b0v + (1 << s1)]
                c0v = iz << (2 * s1)
                cv = [c0v, c0v + (1 << (2 * s1))]
                for k in range(8):
                    h = (a[(k >> 2) & 1] + bv[(k >> 1) & 1] + cv[k & 1]) & mask
                    reg = (l - 1) * 8 + k if l >= 1 else 120 + k
                    if l >= 1:
                        idx_b[pl.ds(reg * C + goff, 16)] = h >> 1
                    else:
                        lix_b[pl.ds(k * C + goff, 16)] = h >> 1
                    par_b[pl.ds(reg * C + goff, 16)] = (h & 1) << 2
                    wgt_b[pl.ds(reg * C + goff, 16)] = w8[k]

            def lbody(li, carry2):
                l = li + 3
                scale = lax.bitcast_convert_type(
                    (l + 131) << 23, jnp.float32) - 1.0
                ix, iy, iz, w8 = frac_weights(px * scale + 0.5,
                                              py * scale + 0.5,
                                              pz * scale + 0.5)
                a = [ix, ix + 1]
                b0v = iy * _P1
                bv = [b0v, b0v + _P1]
                c0v = iz * _P2
                cv = [c0v, c0v + _P2]
                rbase = ((li + 2) * 8) * C + goff
                for k in range(8):
                    h = (a[(k >> 2) & 1] ^ bv[(k >> 1) & 1] ^ cv[k & 1]) \
                        & _XMASK
                    idx_b[pl.ds(rbase + k * C, 16)] = h >> 1
                    par_b[pl.ds(rbase + k * C, 16)] = (h & 1) << 2
                    wgt_b[pl.ds(rbase + k * C, 16)] = w8[k]
                return carry2

            lax.fori_loop(0, 13, lbody, 0)
            return carry

        lax.fori_loop(0, G, body, 0)

    def fire(b):
        pltpu.async_copy(table_h.at[idxs[b]], gaths[b], sems[b])

    def wait(b):
        pltpu.make_async_copy(table_h.at[idxs[b]], gaths[b], sems[b]).wait()

    def out_wait(b):
        pltpu.make_async_copy(
            outs[b],
            out_h.at[pl.ds(base // 8, C // 8), :, pl.ds(0, NLVL * F)],
            osems[b]).wait()

    def combine(ci, b):
        gath_b, lix_b, wgt_b, par_b = gaths[b], lixs[b], wgts[b], pars[b]
        out_b = outs[b]

        @pl.when(ci >= 2)
        def _():
            out_wait(b)

        def body(g, carry):
            goff = g * 16
            tile0 = iota_t + (goff >> 3)

            def emit(acc, col0):
                for f in range(F):
                    plsc.store_scatter(
                        out_b, [tile0, iota_s, jnp.full((16,), 0, jnp.int32)
                                + (col0 + f)], acc[f])

            acc = [None] * F
            for k in range(8):
                rb = (120 + k) * C + goff
                wk = wgt_b[pl.ds(rb, 16)]
                pk = par_b[pl.ds(rb, 16)]
                rows = lix_b[pl.ds(k * C + goff, 16)]
                for f in range(F):
                    r = plsc.load_gather(lvl0, [rows, pk + f])
                    term = wk * r
                    acc[f] = term if k == 0 else acc[f] + term
            emit(acc, 0)

            def lbody(li, carry2):
                rb0 = (li * 8) * C + goff
                acc = [None] * F
                for k in range(8):
                    rb = rb0 + k * C
                    wk = wgt_b[pl.ds(rb, 16)]
                    pk = par_b[pl.ds(rb, 16)]
                    rows = iota + rb
                    for f in range(F):
                        r = plsc.load_gather(gath_b, [rows, pk + f])
                        term = wk * r
                        acc[f] = term if k == 0 else acc[f] + term
                emit(acc, (li + 1) * F)
                return carry2

            lax.fori_loop(0, NLVL - 1, lbody, 0)
            return carry

        lax.fori_loop(0, G, body, 0)
        pltpu.async_copy(out_b,
                         out_h.at[pl.ds((base + ci * C) // 8, C // 8), :,
                                  pl.ds(0, NLVL * F)],
                         osems[b])

    phase_a(0, 0)
    fire(0)

    def outer(oi, carry):
        ci0 = oi * 2
        for b in range(2):
            ci = ci0 + b
            nxt = ci + 1

            @pl.when(nxt < NCHUNK)
            def _():
                phase_a(nxt, 1 - b)
                fire(1 - b)

            wait(b)
            combine(ci, b)
        return carry

    lax.fori_loop(0, NCHUNK // 2, outer, 0)
    out_wait(0)
    out_wait(1)


def kernel(coords, params):
    coords = coords.astype(jnp.float32)
    cx = coords[:, 0]
    cy = coords[:, 1]
    cz = coords[:, 2]
    table = params.reshape(-1, D)
    padded = _hashgrid_sc(cx, cy, cz, table)
    return padded[:, :, : NLVL * F].reshape(N_POINTS, NLVL * F)
